# baseline (speedup 1.0000x reference)
import functools

import jax
import jax.numpy as jnp
from jax import lax
from jax.experimental import pallas as pl
from jax.experimental.pallas import tpu as pltpu
from jax.experimental.pallas import tpu_sc as plsc

_ORDER = 16
_LANES = 16


def _build_sc_gather(n: int, num_workers: int, chunk: int):
    per_w = n // num_workers
    nchunks = per_w // chunk

    mesh = plsc.VectorSubcoreMesh(core_axis_name="c", subcore_axis_name="s")

    @functools.partial(
        pl.kernel,
        mesh=mesh,
        out_type=jax.ShapeDtypeStruct((n,), jnp.float32),
        scratch_types=[
            pltpu.VMEM((_ORDER,), jnp.float32),
            pltpu.VMEM((chunk,), jnp.int32),
            pltpu.VMEM((chunk,), jnp.float32),
        ],
        compiler_params=pltpu.CompilerParams(needs_layout_passes=False),
    )
    def sc_gather(idx_hbm, tbl_hbm, out_hbm, tbl_v, idx_v, out_v):
        wid = lax.axis_index("s") * 2 + lax.axis_index("c")
        base = wid * per_w
        pltpu.sync_copy(tbl_hbm, tbl_v)

        def chunk_body(c, carry):
            off = base + c * chunk
            pltpu.sync_copy(idx_hbm.at[pl.ds(off, chunk)], idx_v)

            def vec_body(i, carry2):
                sl = pl.ds(i * _LANES, _LANES)
                out_v[sl] = plsc.load_gather(tbl_v, [idx_v[sl]])
                return carry2

            lax.fori_loop(0, chunk // _LANES, vec_body, 0, unroll=8)
            pltpu.sync_copy(out_v, out_hbm.at[pl.ds(off, chunk)])
            return carry

        lax.fori_loop(0, nchunks, chunk_body, 0)

    return sc_gather


def kernel(input, val_table):
    b, h = input.shape
    n = b * h
    fn = _build_sc_gather(n, num_workers=32, chunk=12800)
    out = fn(input.reshape(-1), val_table)
    return out.reshape(b, h)


# speedup vs baseline: 1.0376x; 1.0376x over previous
import functools

import jax
import jax.numpy as jnp
from jax import lax
from jax.experimental import pallas as pl
from jax.experimental.pallas import tpu as pltpu
from jax.experimental.pallas import tpu_sc as plsc

_ORDER = 16
_LANES = 16


def _build_sc_gather(n: int, num_workers: int, chunk: int):
    per_w = n // num_workers
    nchunks = per_w // chunk

    mesh = plsc.VectorSubcoreMesh(core_axis_name="c", subcore_axis_name="s")

    @functools.partial(
        pl.kernel,
        mesh=mesh,
        out_type=jax.ShapeDtypeStruct((n,), jnp.float32),
        scratch_types=[
            pltpu.VMEM((_ORDER,), jnp.float32),
            pltpu.VMEM((2, chunk), jnp.int32),
            pltpu.VMEM((2, chunk), jnp.float32),
            pltpu.SemaphoreType.DMA((2,)),
            pltpu.SemaphoreType.DMA((2,)),
        ],
        compiler_params=pltpu.CompilerParams(needs_layout_passes=False),
    )
    def sc_gather(idx_hbm, tbl_hbm, out_hbm, tbl_v, idx_v, out_v, isem, osem):
        wid = lax.axis_index("s") * 2 + lax.axis_index("c")
        base = wid * per_w
        pltpu.sync_copy(tbl_hbm, tbl_v)

        def idx_copy(c, buf):
            return pltpu.make_async_copy(
                idx_hbm.at[pl.ds(base + c * chunk, chunk)],
                idx_v.at[buf],
                isem.at[buf],
            )

        def out_copy(c, buf):
            return pltpu.make_async_copy(
                out_v.at[buf],
                out_hbm.at[pl.ds(base + c * chunk, chunk)],
                osem.at[buf],
            )

        idx_copy(0, 0).start()
        for c in range(nchunks):
            buf = c % 2
            if c + 1 < nchunks:
                idx_copy(c + 1, 1 - buf).start()
            idx_copy(c, buf).wait()
            if c >= 2:
                out_copy(c - 2, buf).wait()

            def vec_body(i, carry):
                sl = pl.ds(i * _LANES, _LANES)
                out_v[buf, sl] = plsc.load_gather(tbl_v, [idx_v[buf, sl]])
                return carry

            lax.fori_loop(0, chunk // _LANES, vec_body, 0, unroll=8)
            out_copy(c, buf).start()
        out_copy(nchunks - 2, nchunks % 2).wait()
        out_copy(nchunks - 1, (nchunks - 1) % 2).wait()

    return sc_gather


def kernel(input, val_table):
    b, h = input.shape
    n = b * h
    fn = _build_sc_gather(n, num_workers=32, chunk=25600)
    out = fn(input.reshape(-1), val_table)
    return out.reshape(b, h)


# speedup vs baseline: 1.4333x; 1.3814x over previous
import functools

import jax
import jax.numpy as jnp
from jax import lax
from jax.experimental import pallas as pl
from jax.experimental.pallas import tpu as pltpu
from jax.experimental.pallas import tpu_sc as plsc

_ORDER = 16
_LANES = 16


def _build_sc_gather(n: int, num_workers: int, chunk: int):
    per_w = n // num_workers
    nchunks = per_w // chunk

    mesh = plsc.VectorSubcoreMesh(core_axis_name="c", subcore_axis_name="s")

    @functools.partial(
        pl.kernel,
        mesh=mesh,
        out_type=jax.ShapeDtypeStruct((n,), jnp.float32),
        scratch_types=[
            pltpu.VMEM((_ORDER,), jnp.float32),
            pltpu.VMEM((2, chunk), jnp.int32),
            pltpu.VMEM((2, chunk), jnp.float32),
            pltpu.SemaphoreType.DMA((2,)),
            pltpu.SemaphoreType.DMA((2,)),
        ],
        compiler_params=pltpu.CompilerParams(needs_layout_passes=False),
    )
    def sc_gather(idx_hbm, tbl_hbm, out_hbm, tbl_v, idx_v, out_v, isem, osem):
        wid = lax.axis_index("s") * 2 + lax.axis_index("c")
        base = wid * per_w
        pltpu.sync_copy(tbl_hbm, tbl_v)

        def idx_copy(c, buf):
            return pltpu.make_async_copy(
                idx_hbm.at[pl.ds(base + c * chunk, chunk)],
                idx_v.at[buf],
                isem.at[buf],
            )

        def out_copy(c, buf):
            return pltpu.make_async_copy(
                out_v.at[buf],
                out_hbm.at[pl.ds(base + c * chunk, chunk)],
                osem.at[buf],
            )

        idx_copy(0, 0).start()
        for c in range(nchunks):
            buf = c % 2
            if c + 1 < nchunks:
                idx_copy(c + 1, 1 - buf).start()
            idx_copy(c, buf).wait()
            if c >= 2:
                out_copy(c - 2, buf).wait()

            @plsc.parallel_loop(0, chunk, _LANES, unroll=8)
            def vec_body(i):
                sl = pl.ds(i, _LANES)
                out_v[buf, sl] = plsc.load_gather(tbl_v, [idx_v[buf, sl]])
            out_copy(c, buf).start()
        out_copy(nchunks - 2, nchunks % 2).wait()
        out_copy(nchunks - 1, (nchunks - 1) % 2).wait()

    return sc_gather


def kernel(input, val_table):
    b, h = input.shape
    n = b * h
    fn = _build_sc_gather(n, num_workers=32, chunk=25600)
    out = fn(input.reshape(-1), val_table)
    return out.reshape(b, h)


# speedup vs baseline: 2.5718x; 1.7943x over previous
import functools

import jax
import jax.numpy as jnp
from jax import lax
from jax.experimental import pallas as pl
from jax.experimental.pallas import tpu as pltpu
from jax.experimental.pallas import tpu_sc as plsc

_ORDER = 16
_LANES = 16


def _build_sc_gather(shape2d, num_workers: int, row_chunk: int):
    rows, cols = shape2d
    rows_per_w = rows // num_workers
    nchunks = rows_per_w // row_chunk
    starts = list(range(0, cols - _LANES + 1, _LANES))
    if starts[-1] + _LANES < cols:
        starts.append(cols - _LANES)

    mesh = plsc.VectorSubcoreMesh(core_axis_name="c", subcore_axis_name="s")

    @functools.partial(
        pl.kernel,
        mesh=mesh,
        out_type=jax.ShapeDtypeStruct(shape2d, jnp.float32),
        scratch_types=[
            pltpu.VMEM((_ORDER,), jnp.float32),
            pltpu.VMEM((2, row_chunk, cols), jnp.int32),
            pltpu.VMEM((2, row_chunk, cols), jnp.float32),
            pltpu.SemaphoreType.DMA((2,)),
            pltpu.SemaphoreType.DMA((2,)),
        ],
        compiler_params=pltpu.CompilerParams(needs_layout_passes=False),
    )
    def sc_gather(idx_hbm, tbl_hbm, out_hbm, tbl_v, idx_v, out_v, isem, osem):
        wid = lax.axis_index("s") * 2 + lax.axis_index("c")
        row0 = wid * rows_per_w
        pltpu.sync_copy(tbl_hbm, tbl_v)

        def idx_copy(c, buf):
            return pltpu.make_async_copy(
                idx_hbm.at[pl.ds(row0 + c * row_chunk, row_chunk), :],
                idx_v.at[buf],
                isem.at[buf],
            )

        def out_copy(c, buf):
            return pltpu.make_async_copy(
                out_v.at[buf],
                out_hbm.at[pl.ds(row0 + c * row_chunk, row_chunk), :],
                osem.at[buf],
            )

        idx_copy(0, 0).start()
        for c in range(nchunks):
            buf = c % 2
            if c + 1 < nchunks:
                idx_copy(c + 1, 1 - buf).start()
            idx_copy(c, buf).wait()
            if c >= 2:
                out_copy(c - 2, buf).wait()

            @plsc.parallel_loop(0, row_chunk, 1, unroll=2)
            def vec_body(r):
                for s in starts:
                    sl = pl.ds(s, _LANES)
                    out_v[buf, r, sl] = plsc.load_gather(
                        tbl_v, [idx_v[buf, r, sl]]
                    )

            out_copy(c, buf).start()
        out_copy(nchunks - 2, nchunks % 2).wait()
        out_copy(nchunks - 1, (nchunks - 1) % 2).wait()

    return sc_gather


def kernel(input, val_table):
    fn = _build_sc_gather(input.shape, num_workers=32, row_chunk=64)
    return fn(input, val_table)


# speedup vs baseline: 2.5777x; 1.0023x over previous
import functools

import jax
import jax.numpy as jnp
from jax import lax
from jax.experimental import pallas as pl
from jax.experimental.pallas import tpu as pltpu
from jax.experimental.pallas import tpu_sc as plsc

_ORDER = 16
_LANES = 16


def _build_sc_gather(shape2d, num_workers: int, row_chunk: int):
    rows, cols = shape2d
    rows_per_w = rows // num_workers
    nchunks = rows_per_w // row_chunk
    starts = list(range(0, cols - _LANES + 1, _LANES))
    if starts[-1] + _LANES < cols:
        starts.append(cols - _LANES)

    mesh = plsc.VectorSubcoreMesh(core_axis_name="c", subcore_axis_name="s")

    @functools.partial(
        pl.kernel,
        mesh=mesh,
        out_type=jax.ShapeDtypeStruct(shape2d, jnp.float32),
        scratch_types=[
            pltpu.VMEM((_ORDER,), jnp.float32),
            pltpu.VMEM((2, row_chunk, cols), jnp.int32),
            pltpu.VMEM((2, row_chunk, cols), jnp.float32),
            pltpu.SemaphoreType.DMA((2,)),
            pltpu.SemaphoreType.DMA((2,)),
        ],
        compiler_params=pltpu.CompilerParams(
            needs_layout_passes=False, skip_device_barrier=True
        ),
    )
    def sc_gather(idx_hbm, tbl_hbm, out_hbm, tbl_v, idx_v, out_v, isem, osem):
        wid = lax.axis_index("s") * 2 + lax.axis_index("c")
        row0 = wid * rows_per_w
        pltpu.sync_copy(tbl_hbm, tbl_v)

        def idx_copy(c, buf):
            return pltpu.make_async_copy(
                idx_hbm.at[pl.ds(row0 + c * row_chunk, row_chunk), :],
                idx_v.at[buf],
                isem.at[buf],
            )

        def out_copy(c, buf):
            return pltpu.make_async_copy(
                out_v.at[buf],
                out_hbm.at[pl.ds(row0 + c * row_chunk, row_chunk), :],
                osem.at[buf],
            )

        idx_copy(0, 0).start()
        for c in range(nchunks):
            buf = c % 2
            if c + 1 < nchunks:
                idx_copy(c + 1, 1 - buf).start()
            idx_copy(c, buf).wait()
            if c >= 2:
                out_copy(c - 2, buf).wait()

            @plsc.parallel_loop(0, row_chunk, 1, unroll=2)
            def vec_body(r):
                for s in starts:
                    sl = pl.ds(s, _LANES)
                    out_v[buf, r, sl] = plsc.load_gather(
                        tbl_v, [idx_v[buf, r, sl]]
                    )

            out_copy(c, buf).start()
        out_copy(nchunks - 2, nchunks % 2).wait()
        out_copy(nchunks - 1, (nchunks - 1) % 2).wait()

    return sc_gather


def kernel(input, val_table):
    fn = _build_sc_gather(input.shape, num_workers=32, row_chunk=64)
    return fn(input, val_table)


# speedup vs baseline: 4.7987x; 1.8616x over previous
import functools

import jax
import jax.numpy as jnp
from jax import lax
from jax.experimental import pallas as pl
from jax.experimental.pallas import tpu as pltpu
from jax.experimental.pallas import tpu_sc as plsc

_ORDER = 16
_LANES = 16


def _build_sc_gather(shape2d, num_workers: int, col_chunk: int):
    rows, cols = shape2d
    cols_per_w = cols // num_workers
    nchunks = cols_per_w // col_chunk
    slices_per_row = col_chunk // _LANES

    mesh = plsc.VectorSubcoreMesh(core_axis_name="c", subcore_axis_name="s")

    @functools.partial(
        pl.kernel,
        mesh=mesh,
        out_type=jax.ShapeDtypeStruct(shape2d, jnp.float32),
        scratch_types=[
            pltpu.VMEM((_ORDER,), jnp.float32),
            pltpu.VMEM((2, rows, col_chunk), jnp.int32),
            pltpu.VMEM((2, rows, col_chunk), jnp.float32),
            pltpu.SemaphoreType.DMA((2,)),
            pltpu.SemaphoreType.DMA((2,)),
        ],
        compiler_params=pltpu.CompilerParams(
            needs_layout_passes=False, skip_device_barrier=True
        ),
    )
    def sc_gather(idx_hbm, tbl_hbm, out_hbm, tbl_v, idx_v, out_v, isem, osem):
        wid = lax.axis_index("s") * 2 + lax.axis_index("c")
        col0 = wid * cols_per_w
        pltpu.sync_copy(tbl_hbm, tbl_v)

        def idx_copy(c, buf):
            return pltpu.make_async_copy(
                idx_hbm.at[:, pl.ds(col0 + c * col_chunk, col_chunk)],
                idx_v.at[buf],
                isem.at[buf],
            )

        def out_copy(c, buf):
            return pltpu.make_async_copy(
                out_v.at[buf],
                out_hbm.at[:, pl.ds(col0 + c * col_chunk, col_chunk)],
                osem.at[buf],
            )

        idx_copy(0, 0).start()
        for c in range(nchunks):
            buf = c % 2
            if c + 1 < nchunks:
                idx_copy(c + 1, 1 - buf).start()
            idx_copy(c, buf).wait()
            if c >= 2:
                out_copy(c - 2, buf).wait()

            @plsc.parallel_loop(0, rows, 1, unroll=2)
            def vec_body(r):
                for j in range(slices_per_row):
                    sl = pl.ds(j * _LANES, _LANES)
                    out_v[buf, r, sl] = plsc.load_gather(
                        tbl_v, [idx_v[buf, r, sl]]
                    )

            out_copy(c, buf).start()
        out_copy(nchunks - 2, nchunks % 2).wait()
        out_copy(nchunks - 1, (nchunks - 1) % 2).wait()

    return sc_gather


def kernel(input, val_table):
    inp_t = input.T
    fn = _build_sc_gather(inp_t.shape, num_workers=32, col_chunk=128)
    out_t = fn(inp_t, val_table)
    return out_t.T


# speedup vs baseline: 5.1461x; 1.0724x over previous
import functools

import jax
import jax.numpy as jnp
from jax import lax
from jax.experimental import pallas as pl
from jax.experimental.pallas import tpu as pltpu
from jax.experimental.pallas import tpu_sc as plsc

_ORDER = 16
_LANES = 16


def _build_sc_gather(shape2d, num_workers: int, col_chunk: int):
    rows, cols = shape2d
    cols_per_w = cols // num_workers
    nchunks = cols_per_w // col_chunk
    slices_per_row = col_chunk // _LANES

    mesh = plsc.VectorSubcoreMesh(core_axis_name="c", subcore_axis_name="s")

    @functools.partial(
        pl.kernel,
        mesh=mesh,
        out_type=jax.ShapeDtypeStruct(shape2d, jnp.float32),
        scratch_types=[
            pltpu.VMEM((_ORDER,), jnp.float32),
            pltpu.VMEM((2, rows, col_chunk), jnp.int32),
            pltpu.VMEM((2, rows, col_chunk), jnp.float32),
            pltpu.SemaphoreType.DMA((2,)),
            pltpu.SemaphoreType.DMA((2,)),
        ],
        compiler_params=pltpu.CompilerParams(
            needs_layout_passes=False, skip_device_barrier=True
        ),
    )
    def sc_gather(idx_hbm, tbl_hbm, out_hbm, tbl_v, idx_v, out_v, isem, osem):
        wid = lax.axis_index("s") * 2 + lax.axis_index("c")
        col0 = wid * cols_per_w
        pltpu.sync_copy(tbl_hbm, tbl_v)
        tbl = tbl_v[...]

        def idx_copy(c, buf):
            return pltpu.make_async_copy(
                idx_hbm.at[:, pl.ds(col0 + c * col_chunk, col_chunk)],
                idx_v.at[buf],
                isem.at[buf],
            )

        def out_copy(c, buf):
            return pltpu.make_async_copy(
                out_v.at[buf],
                out_hbm.at[:, pl.ds(col0 + c * col_chunk, col_chunk)],
                osem.at[buf],
            )

        idx_copy(0, 0).start()
        for c in range(nchunks):
            buf = c % 2
            if c + 1 < nchunks:
                idx_copy(c + 1, 1 - buf).start()
            idx_copy(c, buf).wait()
            if c >= 2:
                out_copy(c - 2, buf).wait()

            @plsc.parallel_loop(0, rows, 1, unroll=2)
            def vec_body(r):
                for j in range(slices_per_row):
                    sl = pl.ds(j * _LANES, _LANES)
                    out_v[buf, r, sl] = jnp.take_along_axis(
                        tbl, idx_v[buf, r, sl], axis=0,
                        mode="promise_in_bounds",
                    )

            out_copy(c, buf).start()
        out_copy(nchunks - 2, nchunks % 2).wait()
        out_copy(nchunks - 1, (nchunks - 1) % 2).wait()

    return sc_gather


def kernel(input, val_table):
    inp_t = input.T
    fn = _build_sc_gather(inp_t.shape, num_workers=32, col_chunk=128)
    out_t = fn(inp_t, val_table)
    return out_t.T
